# baseline (speedup 1.0000x reference)
import dataclasses

import numpy as np

import jax
import jax.numpy as jnp
from jax import lax
from jax.experimental import pallas as pl
from jax.experimental.pallas import tpu as pltpu
from jax.experimental.pallas import tpu_sc as plsc

B, T, FX = 256, 64, 128
INFO = 256
EM_IN = 128
OH_OUT = 64
N_LAYER = 1024
NUM_LAYERS = 4
K = 512

NR = NUM_LAYERS * B
NW = 32
RPW = NR // NW
GRP = 8
NCH = N_LAYER // 16
INT_MIN32 = np.int32(-2147483648)


def _tc_body(xT_ref, e_ref, Wih_ref, Whh_ref, b_ref,
             m1_ref, mb1_ref, m2_ref, mb2_ref,
             g1e_ref, g1t_ref, gb1_ref, g2_ref, gb2_ref, g3_ref, gb3_ref,
             G_ref, out_ref, h_ref, c_ref):
    h_ref[...] = jnp.zeros((B, INFO), jnp.float32)
    c_ref[...] = jnp.zeros((B, INFO), jnp.float32)

    def step(t, carry):
        xt = xT_ref[t]
        gates = (jnp.dot(xt, Wih_ref[...], preferred_element_type=jnp.float32)
                 + jnp.dot(h_ref[...], Whh_ref[...], preferred_element_type=jnp.float32)
                 + b_ref[...])
        i = jax.nn.sigmoid(gates[:, :INFO])
        f = jax.nn.sigmoid(gates[:, INFO:2 * INFO])
        g = jnp.tanh(gates[:, 2 * INFO:3 * INFO])
        o = jax.nn.sigmoid(gates[:, 3 * INFO:])
        c = f * c_ref[...] + i * g
        c_ref[...] = c
        h_ref[...] = o * jnp.tanh(c)
        return carry

    lax.fori_loop(0, T, step, 0)
    traj = h_ref[...]

    emb = (jnp.dot(
        jax.nn.relu(jnp.dot(e_ref[...], m1_ref[...],
                            preferred_element_type=jnp.float32) + mb1_ref[...]),
        m2_ref[...], preferred_element_type=jnp.float32) + mb2_ref[...])

    h1 = jax.nn.relu(
        jnp.dot(emb, g1e_ref[...], preferred_element_type=jnp.float32)
        + jnp.dot(traj, g1t_ref[...], preferred_element_type=jnp.float32)
        + gb1_ref[...])
    h2 = jax.nn.relu(
        jnp.dot(h1, g2_ref[...], preferred_element_type=jnp.float32) + gb2_ref[...])
    mv = jnp.dot(h2, g3_ref[...], preferred_element_type=jnp.float32) + gb3_ref[...]

    for li in range(NUM_LAYERS):
        out_ref[li] = mv[:, li * N_LAYER:(li + 1) * N_LAYER] + G_ref[li]


def _sc_topk_body(z_ref, out_ref, zbuf, keys, hist, cand, obuf):
    wid = lax.axis_index("s") * 2 + lax.axis_index("c")
    base_row = wid * RPW
    lanes = lax.iota(jnp.int32, 16)
    lane_base = lanes * 256
    ones16 = jnp.ones((16,), jnp.int32)
    zeros16 = jnp.zeros((16,), jnp.int32)

    @pl.loop(0, RPW // GRP)
    def _group(gi):
        row0 = base_row + gi * GRP
        pltpu.sync_copy(z_ref.at[pl.ds(row0, GRP)], zbuf)

        @pl.loop(0, GRP)
        def _row(r):
            @pl.loop(0, 16)
            def _z(i):
                for jj in range(16):
                    hist[pl.ds(i * 256 + jj * 16, 16)] = zeros16

            @pl.loop(0, NCH)
            def _a(ci):
                v = zbuf[r, pl.ds(ci * 16, 16)]
                u = plsc.bitcast(v, jnp.int32)
                key = u ^ (lax.shift_right_arithmetic(u, 31)
                           & jnp.int32(0x7FFFFFFF))
                keys[pl.ds(ci * 16, 16)] = key
                digit = lax.shift_right_logical(key ^ INT_MIN32, 24)
                plsc.addupdate_scatter(hist, [lane_base + digit], ones16)

            def _grp_step(j, carry):
                acc, bstar, cgt_d, found = carry
                g = 15 - j
                gv = hist[pl.ds(g * 16, 16)]
                for s in range(1, 16):
                    gv = gv + hist[pl.ds(s * 256 + g * 16, 16)]
                rv = lax.rev(gv, (0,))
                sc = plsc.cumsum(rv)
                Sv = sc + acc
                gtot = jnp.sum(gv)
                anyhit = (acc + gtot) >= K
                hit = Sv >= K
                l = 16 - jnp.max(plsc.all_reduce_population_count(hit))
                rv_l = jnp.sum(jnp.where(lanes == l, rv, 0))
                Sv_l = jnp.sum(jnp.where(lanes == l, Sv, 0))
                take = anyhit & (found == 0)
                bstar = jnp.where(take, g * 16 + 15 - l, bstar)
                cgt_d = jnp.where(take, Sv_l - rv_l, cgt_d)
                found = jnp.where(anyhit, jnp.int32(1), found)
                return acc + gtot, bstar, cgt_d, found

            acc, bstar, cgt_d, found = lax.fori_loop(
                0, 16, _grp_step,
                (jnp.int32(0), jnp.int32(0), jnp.int32(0), jnp.int32(0)))
            kneed2 = K - cgt_d

            def _c_step(ci, nc):
                key = keys[pl.ds(ci * 16, 16)]
                digit = lax.shift_right_logical(key ^ INT_MIN32, 24)
                m = digit == bstar
                low = key & jnp.int32(0x00FFFFFF)
                plsc.store_compressed(cand.at[pl.ds(nc, 16)], low, mask=m)
                return nc + jnp.sum(jnp.where(m, 1, 0))

            nc = lax.fori_loop(0, NCH, _c_step, jnp.int32(0))
            nchunks = (nc + 15) // 16

            def _d_bit(j, carry):
                thr_l, bit = carry
                t = thr_l + bit

                def _d_ch(c2, cv):
                    lowv = cand[pl.ds(c2 * 16, 16)]
                    valid = (c2 * 16 + lanes) < nc
                    return cv + jnp.where(valid & (lowv >= t), 1, 0)

                cvec = lax.fori_loop(0, nchunks, _d_ch, zeros16)
                cnt = jnp.sum(cvec)
                return (jnp.where(cnt >= kneed2, t, thr_l),
                        lax.shift_right_logical(bit, 1))

            thr_l, _ = lax.fori_loop(0, 24, _d_bit,
                                     (jnp.int32(0), jnp.int32(1 << 23)))

            def _d2_ch(c2, cv):
                lowv = cand[pl.ds(c2 * 16, 16)]
                valid = (c2 * 16 + lanes) < nc
                return cv + jnp.where(valid & (lowv > thr_l), 1, 0)

            cgt_low = jnp.sum(lax.fori_loop(0, nchunks, _d2_ch, zeros16))
            need_ties = kneed2 - cgt_low
            thr_key = ((bstar ^ jnp.int32(0x80)) << 24) | thr_l

            def _e_step(ci, tie_base):
                key = keys[pl.ds(ci * 16, 16)]
                gt = key > thr_key
                eq = key == thr_key
                e1 = jnp.where(eq, 1, 0)
                ranks = plsc.cumsum(e1) + tie_base
                sel = gt | (eq & (ranks <= need_ties))
                obuf[r, pl.ds(ci * 16, 16)] = jnp.where(
                    sel, jnp.float32(1.0), jnp.float32(0.0))
                return tie_base + jnp.sum(e1)

            lax.fori_loop(0, NCH, _e_step, jnp.int32(0))

        pltpu.sync_copy(obuf, out_ref.at[pl.ds(row0, GRP)])


def _sc_topk(z2):
    mesh = plsc.VectorSubcoreMesh(core_axis_name="c", subcore_axis_name="s")
    cp = pltpu.CompilerParams()
    if "needs_layout_passes" in pltpu.CompilerParams.__dataclass_fields__:
        cp = dataclasses.replace(cp, needs_layout_passes=False)
    kern = pl.kernel(
        _sc_topk_body,
        out_type=jax.ShapeDtypeStruct((NR, N_LAYER), jnp.float32),
        mesh=mesh,
        compiler_params=cp,
        scratch_types=[
            pltpu.VMEM((GRP, N_LAYER), jnp.float32),
            pltpu.VMEM((N_LAYER,), jnp.int32),
            pltpu.VMEM((16 * 256,), jnp.int32),
            pltpu.VMEM((N_LAYER + 16,), jnp.int32),
            pltpu.VMEM((GRP, N_LAYER), jnp.float32),
        ],
    )
    return kern(z2)


def kernel(x, embedding_input, W_ih, W_hh, b_ih, b_hh,
           mlp_w1, mlp_b1, mlp_w2, mlp_b2,
           g_w1, g_b1, g_w2, g_b2, g_w3, g_b3):
    xT = jnp.swapaxes(x, 0, 1)
    e = jnp.squeeze(embedding_input, axis=1)
    b = (b_ih + b_hh).reshape(1, 4 * INFO)
    g1e = g_w1[:OH_OUT]
    g1t = g_w1[OH_OUT:]

    gkey = jax.random.key(42)
    G = jnp.stack([
        jax.random.gumbel(jax.random.fold_in(gkey, li), (B, N_LAYER), jnp.float32)
        for li in range(NUM_LAYERS)
    ], axis=0)

    z = pl.pallas_call(
        _tc_body,
        out_shape=jax.ShapeDtypeStruct((NUM_LAYERS, B, N_LAYER), jnp.float32),
        scratch_shapes=[
            pltpu.VMEM((B, INFO), jnp.float32),
            pltpu.VMEM((B, INFO), jnp.float32),
        ],
    )(xT, e, W_ih, W_hh, b,
      mlp_w1, mlp_b1.reshape(1, -1), mlp_w2, mlp_b2.reshape(1, -1),
      g1e, g1t, g_b1.reshape(1, -1), g_w2, g_b2.reshape(1, -1), g_w3,
      g_b3.reshape(1, -1), G)

    masks2 = _sc_topk(z.reshape(NR, N_LAYER))
    return masks2.reshape(NUM_LAYERS, B, N_LAYER)


# speedup vs baseline: 1.1012x; 1.1012x over previous
import dataclasses

import numpy as np

import jax
import jax.numpy as jnp
from jax import lax
from jax.experimental import pallas as pl
from jax.experimental.pallas import tpu as pltpu
from jax.experimental.pallas import tpu_sc as plsc

B, T, FX = 256, 64, 128
INFO = 256
EM_IN = 128
OH_OUT = 64
N_LAYER = 1024
NUM_LAYERS = 4
K = 512

NR = NUM_LAYERS * B
NW = 32
RPW = NR // NW
GRP = 8
NCH = N_LAYER // 16
INT_MIN32 = np.int32(-2147483648)


def _tc_body(xT_ref, e_ref, Wih_ref, Whh_ref, b_ref,
             m1_ref, mb1_ref, m2_ref, mb2_ref,
             g1e_ref, g1t_ref, gb1_ref, g2_ref, gb2_ref, g3_ref, gb3_ref,
             G_ref, out_ref, h_ref, c_ref):
    h_ref[...] = jnp.zeros((B, INFO), jnp.float32)
    c_ref[...] = jnp.zeros((B, INFO), jnp.float32)

    def step(t, carry):
        xt = xT_ref[t]
        gates = (jnp.dot(xt, Wih_ref[...], preferred_element_type=jnp.float32)
                 + jnp.dot(h_ref[...], Whh_ref[...], preferred_element_type=jnp.float32)
                 + b_ref[...])
        i = jax.nn.sigmoid(gates[:, :INFO])
        f = jax.nn.sigmoid(gates[:, INFO:2 * INFO])
        g = jnp.tanh(gates[:, 2 * INFO:3 * INFO])
        o = jax.nn.sigmoid(gates[:, 3 * INFO:])
        c = f * c_ref[...] + i * g
        c_ref[...] = c
        h_ref[...] = o * jnp.tanh(c)
        return carry

    lax.fori_loop(0, T, step, 0)
    traj = h_ref[...]

    emb = (jnp.dot(
        jax.nn.relu(jnp.dot(e_ref[...], m1_ref[...],
                            preferred_element_type=jnp.float32) + mb1_ref[...]),
        m2_ref[...], preferred_element_type=jnp.float32) + mb2_ref[...])

    h1 = jax.nn.relu(
        jnp.dot(emb, g1e_ref[...], preferred_element_type=jnp.float32)
        + jnp.dot(traj, g1t_ref[...], preferred_element_type=jnp.float32)
        + gb1_ref[...])
    h2 = jax.nn.relu(
        jnp.dot(h1, g2_ref[...], preferred_element_type=jnp.float32) + gb2_ref[...])
    mv = jnp.dot(h2, g3_ref[...], preferred_element_type=jnp.float32) + gb3_ref[...]

    for li in range(NUM_LAYERS):
        out_ref[li] = mv[:, li * N_LAYER:(li + 1) * N_LAYER] + G_ref[li]


def _sc_topk_body(z_ref, out_ref, zbuf, ukeys, hist, obuf):
    wid = lax.axis_index("s") * 2 + lax.axis_index("c")
    lanes = lax.iota(jnp.int32, 16)
    ones16 = jnp.ones((16,), jnp.int32)
    zeros16 = jnp.zeros((16,), jnp.int32)
    kvec = jnp.full((16,), K, jnp.int32)

    @pl.loop(0, 2)
    def _task(ti):
        row0 = (wid + ti * NW) * 16
        pltpu.sync_copy(z_ref.at[pl.ds(row0, 16)], zbuf)

        @pl.loop(0, 256)
        def _z(i):
            for h in range(4):
                hist[pl.ds(h * 4096 + i * 16, 16)] = zeros16

        @pl.loop(0, N_LAYER, step=4)
        def _k(j):
            for u in range(4):
                jv = jnp.full((16,), j + u, jnp.int32)
                v = plsc.load_gather(zbuf, [lanes, jv])
                bts = plsc.bitcast(v, jnp.int32)
                key = bts ^ (lax.shift_right_arithmetic(bts, 31)
                             & jnp.int32(0x7FFFFFFF))
                uk = key ^ INT_MIN32
                ukeys[pl.ds((j + u) * 16, 16)] = uk
                dig = lax.shift_right_logical(uk, 24)
                plsc.addupdate_scatter(hist, [dig * 16 + lanes], ones16)

        def _scan(level, kneed_v):
            def _s(jj, carry):
                acc, bstar, cgtl, found = carry
                for u in range(4):
                    b = 255 - (jj * 4 + u)
                    h = hist[pl.ds(level * 4096 + b * 16, 16)]
                    acc2 = acc + h
                    hit = jnp.logical_and(jnp.logical_not(found),
                                          acc2 >= kneed_v)
                    bstar = jnp.where(hit, b, bstar)
                    cgtl = jnp.where(hit, acc, cgtl)
                    found = jnp.logical_or(found, acc2 >= kneed_v)
                    acc = acc2
                return acc, bstar, cgtl, found

            _, bstar, cgtl, _ = lax.fori_loop(
                0, 64, _s,
                (zeros16, zeros16, zeros16, jnp.zeros((16,), jnp.bool_)))
            return bstar, cgtl

        bstar, cgtl = _scan(0, kvec)
        prefix_v = bstar
        kneed_v = kvec - cgtl

        for level, s in ((1, 16), (2, 8), (3, 0)):
            @pl.loop(0, N_LAYER, step=4)
            def _l(j, _s_=s, _lv_=level, _pv_=prefix_v):
                for u in range(4):
                    uk = ukeys[pl.ds((j + u) * 16, 16)]
                    act = lax.shift_right_logical(uk, _s_ + 8) == _pv_
                    dig = (lax.shift_right_logical(uk, _s_)
                           & jnp.int32(0xFF))
                    plsc.addupdate_scatter(
                        hist, [_lv_ * 4096 + dig * 16 + lanes], ones16,
                        mask=act)

            bstar, cgtl = _scan(level, kneed_v)
            prefix_v = (prefix_v << 8) | bstar
            kneed_v = kneed_v - cgtl

        thr_x = prefix_v ^ INT_MIN32

        def _f(jj, rank):
            j = jj * 4
            for u in range(4):
                uk = ukeys[pl.ds((j + u) * 16, 16)]
                kx = uk ^ INT_MIN32
                gt = kx > thr_x
                eq = uk == prefix_v
                rank = rank + jnp.where(eq, 1, 0)
                sel = jnp.logical_or(gt, jnp.logical_and(eq, rank <= kneed_v))
                val = jnp.where(sel, jnp.float32(1.0), jnp.float32(0.0))
                jv = jnp.full((16,), j + u, jnp.int32)
                plsc.store_scatter(obuf, [lanes, jv], val)
            return rank

        lax.fori_loop(0, 256, _f, zeros16)
        pltpu.sync_copy(obuf, out_ref.at[pl.ds(row0, 16)])


def _sc_topk(z2):
    mesh = plsc.VectorSubcoreMesh(core_axis_name="c", subcore_axis_name="s")
    cp = pltpu.CompilerParams()
    if "needs_layout_passes" in pltpu.CompilerParams.__dataclass_fields__:
        cp = dataclasses.replace(cp, needs_layout_passes=False)
    kern = pl.kernel(
        _sc_topk_body,
        out_type=jax.ShapeDtypeStruct((NR, N_LAYER), jnp.float32),
        mesh=mesh,
        compiler_params=cp,
        scratch_types=[
            pltpu.VMEM((16, N_LAYER), jnp.float32),
            pltpu.VMEM((N_LAYER * 16,), jnp.int32),
            pltpu.VMEM((4 * 4096,), jnp.int32),
            pltpu.VMEM((16, N_LAYER), jnp.float32),
        ],
    )
    return kern(z2)


def kernel(x, embedding_input, W_ih, W_hh, b_ih, b_hh,
           mlp_w1, mlp_b1, mlp_w2, mlp_b2,
           g_w1, g_b1, g_w2, g_b2, g_w3, g_b3):
    xT = jnp.swapaxes(x, 0, 1)
    e = jnp.squeeze(embedding_input, axis=1)
    b = (b_ih + b_hh).reshape(1, 4 * INFO)
    g1e = g_w1[:OH_OUT]
    g1t = g_w1[OH_OUT:]

    gkey = jax.random.key(42)
    G = jnp.stack([
        jax.random.gumbel(jax.random.fold_in(gkey, li), (B, N_LAYER), jnp.float32)
        for li in range(NUM_LAYERS)
    ], axis=0)

    z = pl.pallas_call(
        _tc_body,
        out_shape=jax.ShapeDtypeStruct((NUM_LAYERS, B, N_LAYER), jnp.float32),
        scratch_shapes=[
            pltpu.VMEM((B, INFO), jnp.float32),
            pltpu.VMEM((B, INFO), jnp.float32),
        ],
    )(xT, e, W_ih, W_hh, b,
      mlp_w1, mlp_b1.reshape(1, -1), mlp_w2, mlp_b2.reshape(1, -1),
      g1e, g1t, g_b1.reshape(1, -1), g_w2, g_b2.reshape(1, -1), g_w3,
      g_b3.reshape(1, -1), G)

    masks2 = _sc_topk(z.reshape(NR, N_LAYER))
    return masks2.reshape(NUM_LAYERS, B, N_LAYER)


# speedup vs baseline: 1.1125x; 1.0103x over previous
import dataclasses

import numpy as np

import jax
import jax.numpy as jnp
from jax import lax
from jax.experimental import pallas as pl
from jax.experimental.pallas import tpu as pltpu
from jax.experimental.pallas import tpu_sc as plsc

B, T, FX = 256, 64, 128
INFO = 256
EM_IN = 128
OH_OUT = 64
N_LAYER = 1024
NUM_LAYERS = 4
K = 512

NR = NUM_LAYERS * B
NW = 32
RPW = NR // NW
GRP = 8
NCH = N_LAYER // 16
INT_MIN32 = np.int32(-2147483648)


def _tc_body(xT_ref, e_ref, Wih_ref, Whh_ref, b_ref,
             m1_ref, mb1_ref, m2_ref, mb2_ref,
             g1e_ref, g1t_ref, gb1_ref, g2_ref, gb2_ref, g3_ref, gb3_ref,
             G_ref, out_ref, h_ref, c_ref):
    h_ref[...] = jnp.zeros((B, INFO), jnp.float32)
    c_ref[...] = jnp.zeros((B, INFO), jnp.float32)

    def step(t, carry):
        xt = xT_ref[t]
        gates = (jnp.dot(xt, Wih_ref[...], preferred_element_type=jnp.float32)
                 + jnp.dot(h_ref[...], Whh_ref[...], preferred_element_type=jnp.float32)
                 + b_ref[...])
        i = jax.nn.sigmoid(gates[:, :INFO])
        f = jax.nn.sigmoid(gates[:, INFO:2 * INFO])
        g = jnp.tanh(gates[:, 2 * INFO:3 * INFO])
        o = jax.nn.sigmoid(gates[:, 3 * INFO:])
        c = f * c_ref[...] + i * g
        c_ref[...] = c
        h_ref[...] = o * jnp.tanh(c)
        return carry

    lax.fori_loop(0, T, step, 0)
    traj = h_ref[...]

    emb = (jnp.dot(
        jax.nn.relu(jnp.dot(e_ref[...], m1_ref[...],
                            preferred_element_type=jnp.float32) + mb1_ref[...]),
        m2_ref[...], preferred_element_type=jnp.float32) + mb2_ref[...])

    h1 = jax.nn.relu(
        jnp.dot(emb, g1e_ref[...], preferred_element_type=jnp.float32)
        + jnp.dot(traj, g1t_ref[...], preferred_element_type=jnp.float32)
        + gb1_ref[...])
    h2 = jax.nn.relu(
        jnp.dot(h1, g2_ref[...], preferred_element_type=jnp.float32) + gb2_ref[...])
    mv = jnp.dot(h2, g3_ref[...], preferred_element_type=jnp.float32) + gb3_ref[...]

    for li in range(NUM_LAYERS):
        out_ref[li] = mv[:, li * N_LAYER:(li + 1) * N_LAYER] + G_ref[li]


def _sc_topk_body(z_ref, out_ref, zbuf, ukeys, hist, obuf):
    wid = lax.axis_index("s") * 2 + lax.axis_index("c")
    lanes = lax.iota(jnp.int32, 16)
    ones16 = jnp.ones((16,), jnp.int32)
    zeros16 = jnp.zeros((16,), jnp.int32)
    kvec = jnp.full((16,), K, jnp.int32)

    @pl.loop(0, 2)
    def _task(ti):
        row0 = (wid + ti * NW) * 16
        pltpu.sync_copy(z_ref.at[pl.ds(row0, 16)], zbuf.at[:, pl.ds(0, N_LAYER)])

        @pl.loop(0, 256)
        def _z(i):
            for h in range(4):
                hist[pl.ds(h * 4096 + i * 16, 16)] = zeros16

        @pl.loop(0, N_LAYER, step=8)
        def _k(j):
            for u in range(8):
                jv = jnp.full((16,), j + u, jnp.int32)
                v = plsc.load_gather(zbuf, [lanes, jv])
                bts = plsc.bitcast(v, jnp.int32)
                key = bts ^ (lax.shift_right_arithmetic(bts, 31)
                             & jnp.int32(0x7FFFFFFF))
                uk = key ^ INT_MIN32
                ukeys[pl.ds((j + u) * 16, 16)] = uk
                dig = lax.shift_right_logical(uk, 24)
                plsc.addupdate_scatter(hist, [dig * 16 + lanes], ones16)

        def _scan(level, kneed_v):
            def _s(jj, carry):
                acc, bstar, cgtl, found = carry
                for u in range(4):
                    b = 255 - (jj * 4 + u)
                    h = hist[pl.ds(level * 4096 + b * 16, 16)]
                    acc2 = acc + h
                    hit = jnp.logical_and(jnp.logical_not(found),
                                          acc2 >= kneed_v)
                    bstar = jnp.where(hit, b, bstar)
                    cgtl = jnp.where(hit, acc, cgtl)
                    found = jnp.logical_or(found, acc2 >= kneed_v)
                    acc = acc2
                return acc, bstar, cgtl, found

            _, bstar, cgtl, _ = lax.fori_loop(
                0, 64, _s,
                (zeros16, zeros16, zeros16, jnp.zeros((16,), jnp.bool_)))
            return bstar, cgtl

        bstar, cgtl = _scan(0, kvec)
        prefix_v = bstar
        kneed_v = kvec - cgtl

        for level, s in ((1, 16), (2, 8), (3, 0)):
            @pl.loop(0, N_LAYER, step=8)
            def _l(j, _s_=s, _lv_=level, _pv_=prefix_v):
                for u in range(8):
                    uk = ukeys[pl.ds((j + u) * 16, 16)]
                    act = lax.shift_right_logical(uk, _s_ + 8) == _pv_
                    dig = (lax.shift_right_logical(uk, _s_)
                           & jnp.int32(0xFF))
                    plsc.addupdate_scatter(
                        hist, [_lv_ * 4096 + dig * 16 + lanes], ones16,
                        mask=act)

            bstar, cgtl = _scan(level, kneed_v)
            prefix_v = (prefix_v << 8) | bstar
            kneed_v = kneed_v - cgtl

        thr_x = prefix_v ^ INT_MIN32

        def _f(jj, rank):
            j = jj * 8
            for u in range(8):
                uk = ukeys[pl.ds((j + u) * 16, 16)]
                kx = uk ^ INT_MIN32
                gt = kx > thr_x
                eq = uk == prefix_v
                rank = rank + jnp.where(eq, 1, 0)
                sel = jnp.logical_or(gt, jnp.logical_and(eq, rank <= kneed_v))
                val = jnp.where(sel, jnp.float32(1.0), jnp.float32(0.0))
                jv = jnp.full((16,), j + u, jnp.int32)
                plsc.store_scatter(obuf, [lanes, jv], val)
            return rank

        lax.fori_loop(0, 128, _f, zeros16)
        pltpu.sync_copy(obuf.at[:, pl.ds(0, N_LAYER)], out_ref.at[pl.ds(row0, 16)])


def _sc_topk(z2):
    mesh = plsc.VectorSubcoreMesh(core_axis_name="c", subcore_axis_name="s")
    cp = pltpu.CompilerParams()
    if "needs_layout_passes" in pltpu.CompilerParams.__dataclass_fields__:
        cp = dataclasses.replace(cp, needs_layout_passes=False)
    kern = pl.kernel(
        _sc_topk_body,
        out_type=jax.ShapeDtypeStruct((NR, N_LAYER), jnp.float32),
        mesh=mesh,
        compiler_params=cp,
        scratch_types=[
            pltpu.VMEM((16, N_LAYER + 1), jnp.float32),
            pltpu.VMEM((N_LAYER * 16,), jnp.int32),
            pltpu.VMEM((4 * 4096,), jnp.int32),
            pltpu.VMEM((16, N_LAYER + 1), jnp.float32),
        ],
    )
    return kern(z2)


def kernel(x, embedding_input, W_ih, W_hh, b_ih, b_hh,
           mlp_w1, mlp_b1, mlp_w2, mlp_b2,
           g_w1, g_b1, g_w2, g_b2, g_w3, g_b3):
    xT = jnp.swapaxes(x, 0, 1)
    e = jnp.squeeze(embedding_input, axis=1)
    b = (b_ih + b_hh).reshape(1, 4 * INFO)
    g1e = g_w1[:OH_OUT]
    g1t = g_w1[OH_OUT:]

    gkey = jax.random.key(42)
    G = jnp.stack([
        jax.random.gumbel(jax.random.fold_in(gkey, li), (B, N_LAYER), jnp.float32)
        for li in range(NUM_LAYERS)
    ], axis=0)

    z = pl.pallas_call(
        _tc_body,
        out_shape=jax.ShapeDtypeStruct((NUM_LAYERS, B, N_LAYER), jnp.float32),
        scratch_shapes=[
            pltpu.VMEM((B, INFO), jnp.float32),
            pltpu.VMEM((B, INFO), jnp.float32),
        ],
    )(xT, e, W_ih, W_hh, b,
      mlp_w1, mlp_b1.reshape(1, -1), mlp_w2, mlp_b2.reshape(1, -1),
      g1e, g1t, g_b1.reshape(1, -1), g_w2, g_b2.reshape(1, -1), g_w3,
      g_b3.reshape(1, -1), G)

    masks2 = _sc_topk(z.reshape(NR, N_LAYER))
    return masks2.reshape(NUM_LAYERS, B, N_LAYER)


# speedup vs baseline: 1.6243x; 1.4601x over previous
import dataclasses

import numpy as np

import jax
import jax.numpy as jnp
from jax import lax
from jax.experimental import pallas as pl
from jax.experimental.pallas import tpu as pltpu
from jax.experimental.pallas import tpu_sc as plsc

B, T, FX = 256, 64, 128
INFO = 256
EM_IN = 128
OH_OUT = 64
N_LAYER = 1024
NUM_LAYERS = 4
K = 512

NR = NUM_LAYERS * B
NW = 32
RPW = NR // NW
GRP = 8
NCH = N_LAYER // 16
INT_MIN32 = np.int32(-2147483648)


def _tc_body(xT_ref, e_ref, Wih_ref, Whh_ref, b_ref,
             m1_ref, mb1_ref, m2_ref, mb2_ref,
             g1e_ref, g1t_ref, gb1_ref, g2_ref, gb2_ref, g3_ref, gb3_ref,
             G_ref, out_ref, h_ref, c_ref):
    h_ref[...] = jnp.zeros((B, INFO), jnp.float32)
    c_ref[...] = jnp.zeros((B, INFO), jnp.float32)

    def step(t, carry):
        xt = xT_ref[t]
        gates = (jnp.dot(xt, Wih_ref[...], preferred_element_type=jnp.float32)
                 + jnp.dot(h_ref[...], Whh_ref[...], preferred_element_type=jnp.float32)
                 + b_ref[...])
        i = jax.nn.sigmoid(gates[:, :INFO])
        f = jax.nn.sigmoid(gates[:, INFO:2 * INFO])
        g = jnp.tanh(gates[:, 2 * INFO:3 * INFO])
        o = jax.nn.sigmoid(gates[:, 3 * INFO:])
        c = f * c_ref[...] + i * g
        c_ref[...] = c
        h_ref[...] = o * jnp.tanh(c)
        return carry

    lax.fori_loop(0, T, step, 0)
    traj = h_ref[...]

    emb = (jnp.dot(
        jax.nn.relu(jnp.dot(e_ref[...], m1_ref[...],
                            preferred_element_type=jnp.float32) + mb1_ref[...]),
        m2_ref[...], preferred_element_type=jnp.float32) + mb2_ref[...])

    h1 = jax.nn.relu(
        jnp.dot(emb, g1e_ref[...], preferred_element_type=jnp.float32)
        + jnp.dot(traj, g1t_ref[...], preferred_element_type=jnp.float32)
        + gb1_ref[...])
    h2 = jax.nn.relu(
        jnp.dot(h1, g2_ref[...], preferred_element_type=jnp.float32) + gb2_ref[...])
    mv = jnp.dot(h2, g3_ref[...], preferred_element_type=jnp.float32) + gb3_ref[...]

    for li in range(NUM_LAYERS):
        out_ref[li] = mv[:, li * N_LAYER:(li + 1) * N_LAYER] + G_ref[li]


def _sc_topk_body(z_ref, out_ref, zbuf, ukeys, hist, obuf):
    wid = lax.axis_index("s") * 2 + lax.axis_index("c")
    lanes = lax.iota(jnp.int32, 16)
    ones16 = jnp.ones((16,), jnp.int32)
    zeros16 = jnp.zeros((16,), jnp.int32)
    kvec = jnp.full((16,), K, jnp.int32)

    @pl.loop(0, 2)
    def _task(ti):
        row0 = (wid + ti * NW) * 16
        pltpu.sync_copy(z_ref.at[pl.ds(row0, 16)], zbuf.at[:, pl.ds(0, N_LAYER)])

        @plsc.parallel_loop(0, 1024, unroll=8)
        def _z(i):
            hist[pl.ds(i * 16, 16)] = zeros16

        @plsc.parallel_loop(0, N_LAYER, unroll=8)
        def _k(j):
            jv = jnp.full((16,), j, jnp.int32)
            v = plsc.load_gather(zbuf, [lanes, jv])
            bts = plsc.bitcast(v, jnp.int32)
            key = bts ^ (lax.shift_right_arithmetic(bts, 31)
                         & jnp.int32(0x7FFFFFFF))
            uk = key ^ INT_MIN32
            ukeys[pl.ds(j * 16, 16)] = uk
            dig = lax.shift_right_logical(uk, 24)
            plsc.addupdate_scatter(hist, [dig * 16 + lanes], ones16)

        def _scan(level, kneed_v):
            @plsc.parallel_loop(
                0, 256, unroll=8,
                carry=(zeros16, zeros16, zeros16,
                       jnp.zeros((16,), jnp.bool_)))
            def _s(i, carry):
                acc, bstar, cgtl, found = carry
                b = 255 - i
                h = hist[pl.ds(level * 4096 + b * 16, 16)]
                acc2 = acc + h
                hit = jnp.logical_and(jnp.logical_not(found),
                                      acc2 >= kneed_v)
                bstar = jnp.where(hit, b, bstar)
                cgtl = jnp.where(hit, acc, cgtl)
                found = jnp.logical_or(found, acc2 >= kneed_v)
                return acc2, bstar, cgtl, found

            _, bstar, cgtl, _ = _s
            return bstar, cgtl

        bstar, cgtl = _scan(0, kvec)
        prefix_v = bstar
        kneed_v = kvec - cgtl

        for level, s in ((1, 16), (2, 8), (3, 0)):
            @plsc.parallel_loop(0, N_LAYER, unroll=8)
            def _l(j, _s_=s, _lv_=level, _pv_=prefix_v):
                uk = ukeys[pl.ds(j * 16, 16)]
                act = lax.shift_right_logical(uk, _s_ + 8) == _pv_
                dig = (lax.shift_right_logical(uk, _s_)
                       & jnp.int32(0xFF))
                plsc.addupdate_scatter(
                    hist, [_lv_ * 4096 + dig * 16 + lanes], ones16,
                    mask=act)

            bstar, cgtl = _scan(level, kneed_v)
            prefix_v = (prefix_v << 8) | bstar
            kneed_v = kneed_v - cgtl

        thr_x = prefix_v ^ INT_MIN32

        @plsc.parallel_loop(0, N_LAYER, unroll=8, carry=zeros16)
        def _f(j, rank):
            uk = ukeys[pl.ds(j * 16, 16)]
            kx = uk ^ INT_MIN32
            gt = kx > thr_x
            eq = uk == prefix_v
            rank = rank + jnp.where(eq, 1, 0)
            sel = jnp.logical_or(gt, jnp.logical_and(eq, rank <= kneed_v))
            val = jnp.where(sel, jnp.float32(1.0), jnp.float32(0.0))
            jv = jnp.full((16,), j, jnp.int32)
            plsc.store_scatter(obuf, [lanes, jv], val)
            return rank
        pltpu.sync_copy(obuf.at[:, pl.ds(0, N_LAYER)], out_ref.at[pl.ds(row0, 16)])


def _sc_topk(z2):
    mesh = plsc.VectorSubcoreMesh(core_axis_name="c", subcore_axis_name="s")
    cp = pltpu.CompilerParams()
    if "needs_layout_passes" in pltpu.CompilerParams.__dataclass_fields__:
        cp = dataclasses.replace(cp, needs_layout_passes=False)
    kern = pl.kernel(
        _sc_topk_body,
        out_type=jax.ShapeDtypeStruct((NR, N_LAYER), jnp.float32),
        mesh=mesh,
        compiler_params=cp,
        scratch_types=[
            pltpu.VMEM((16, N_LAYER + 1), jnp.float32),
            pltpu.VMEM((N_LAYER * 16,), jnp.int32),
            pltpu.VMEM((4 * 4096,), jnp.int32),
            pltpu.VMEM((16, N_LAYER + 1), jnp.float32),
        ],
    )
    return kern(z2)


def kernel(x, embedding_input, W_ih, W_hh, b_ih, b_hh,
           mlp_w1, mlp_b1, mlp_w2, mlp_b2,
           g_w1, g_b1, g_w2, g_b2, g_w3, g_b3):
    xT = jnp.swapaxes(x, 0, 1)
    e = jnp.squeeze(embedding_input, axis=1)
    b = (b_ih + b_hh).reshape(1, 4 * INFO)
    g1e = g_w1[:OH_OUT]
    g1t = g_w1[OH_OUT:]

    gkey = jax.random.key(42)
    G = jnp.stack([
        jax.random.gumbel(jax.random.fold_in(gkey, li), (B, N_LAYER), jnp.float32)
        for li in range(NUM_LAYERS)
    ], axis=0)

    z = pl.pallas_call(
        _tc_body,
        out_shape=jax.ShapeDtypeStruct((NUM_LAYERS, B, N_LAYER), jnp.float32),
        scratch_shapes=[
            pltpu.VMEM((B, INFO), jnp.float32),
            pltpu.VMEM((B, INFO), jnp.float32),
        ],
    )(xT, e, W_ih, W_hh, b,
      mlp_w1, mlp_b1.reshape(1, -1), mlp_w2, mlp_b2.reshape(1, -1),
      g1e, g1t, g_b1.reshape(1, -1), g_w2, g_b2.reshape(1, -1), g_w3,
      g_b3.reshape(1, -1), G)

    masks2 = _sc_topk(z.reshape(NR, N_LAYER))
    return masks2.reshape(NUM_LAYERS, B, N_LAYER)
